# baseline (speedup 1.0000x reference)
import functools

import jax
import jax.numpy as jnp
from jax import lax
from jax.experimental import pallas as pl
from jax.experimental.pallas import tpu as pltpu
from jax.experimental.pallas import tpu_sc as plsc

N = 10000
NP = 10240
E = 320000
EP = 344064
NREL = 1000
RP = 1024
D = 128
NC = 2
NS = 16
NW = NC * NS
EPT = EP // NW
CH0 = 128
NCHUNK0 = EP // NS // CH0
CHL = 56
NCHUNKL = EPT // CHL
NPA = 10112
RPS = NPA // NS
RPSD = NP // NS
DUMMY = N + 1

_MESH = plsc.VectorSubcoreMesh(core_axis_name="c", subcore_axis_name="s")
_SC_PARAMS = pltpu.CompilerParams(needs_layout_passes=False)
f32 = jnp.float32
i32 = jnp.int32


def _pass0_body(ctab, epack, zrows, zvec,
                acc_out, den_out,
                pack0, pack1, cidx0, cidx1, ones_v, rows0, rows1,
                acc_sh, den_sh, sg0, sg1, ss0, ss1):
    cid = lax.axis_index("c")
    sid = lax.axis_index("s")
    pltpu.sync_copy(zrows.at[pl.ds(sid * RPS, RPS)],
                    acc_sh.at[pl.ds(sid * RPS, RPS)])
    pltpu.sync_copy(zvec.at[pl.ds(sid * RPSD, RPSD)],
                    den_sh.at[pl.ds(sid * RPSD, RPSD)])
    for k in range(CH0 // 16):
        ones_v[pl.ds(k * 16, 16)] = jnp.ones((16,), f32)
    plsc.subcore_barrier()

    is_ent = lax.broadcast(cid == 0, (16,))
    base = sid * NCHUNK0
    bufs = ((pack0, cidx0, rows0, sg0, ss0),
            (pack1, cidx1, rows1, sg1, ss1))

    def fire_gathers(b, c):
        pack, cv, rows, sg, _ = bufs[b]
        pltpu.sync_copy(epack.at[c], pack)
        for k in range(CH0 // 16):
            sl = pl.ds(k * 16, 16)
            cv[sl] = jnp.where(is_ent, pack[0, sl], pack[2, sl] + N)
        pltpu.async_copy(ctab.at[cv], rows, sg)

    def wait_gathers(b):
        pack, cv, rows, sg, _ = bufs[b]
        pltpu.make_async_copy(ctab.at[cv], rows, sg).wait()

    def fire_scatters(b):
        pack, cv, rows, _, ss = bufs[b]
        d1 = pltpu.async_copy(rows, acc_sh.at[pack.at[1]], ss, add=True)
        d2 = pltpu.async_copy(ones_v, den_sh.at[pack.at[1]], ss, add=True)
        return d1, d2

    fire_gathers(0, base)
    fire_gathers(1, base + 1)

    def pair(i, carry):
        c0 = 2 * i
        wait_gathers(0)
        s0 = fire_scatters(0)
        wait_gathers(1)
        s1 = fire_scatters(1)
        s0[0].wait()
        s0[1].wait()
        fire_gathers(0, base + jnp.minimum(c0 + 2, NCHUNK0 - 1))
        s1[0].wait()
        s1[1].wait()
        fire_gathers(1, base + jnp.minimum(c0 + 3, NCHUNK0 - 1))
        return carry

    lax.fori_loop(0, NCHUNK0 // 2, pair, 0)
    wait_gathers(0)
    wait_gathers(1)
    plsc.subcore_barrier()
    pltpu.sync_copy(acc_sh.at[pl.ds(sid * RPS, RPS)],
                    acc_out.at[cid, pl.ds(sid * RPS, RPS)])
    pltpu.sync_copy(den_sh.at[pl.ds(sid * RPSD, RPSD)],
                    den_out.at[cid, pl.ds(sid * RPSD, RPSD)])


_pass0 = pl.kernel(
    _pass0_body,
    out_type=[jax.ShapeDtypeStruct((NC, NP, D), f32),
              jax.ShapeDtypeStruct((NC, NP), f32)],
    mesh=_MESH,
    compiler_params=_SC_PARAMS,
    scratch_types=[
        pltpu.VMEM((4, CH0), i32),
        pltpu.VMEM((4, CH0), i32),
        pltpu.VMEM((CH0,), i32),
        pltpu.VMEM((CH0,), i32),
        pltpu.VMEM((CH0,), f32),
        pltpu.VMEM((CH0, D), f32),
        pltpu.VMEM((CH0, D), f32),
        pltpu.VMEM_SHARED((NPA, D), f32),
        pltpu.VMEM_SHARED((NP,), f32),
        pltpu.SemaphoreType.DMA,
        pltpu.SemaphoreType.DMA,
        pltpu.SemaphoreType.DMA,
        pltpu.SemaphoreType.DMA,
    ],
)


def _layer_body(packl, dstp, eflat, qflat, fmat, rn, zrows, zvec,
                acc_out, den_out,
                pack0, pack1, d0, d1, e0, e1, q0, q1, f0, f1, r0, r1, m0, m1,
                acc_sh, den_sh, sg0, sg1, ss0, ss1):
    cid = lax.axis_index("c")
    sid = lax.axis_index("s")
    wid = sid * NC + cid
    pltpu.sync_copy(zrows.at[pl.ds(sid * RPS, RPS)],
                    acc_sh.at[pl.ds(sid * RPS, RPS)])
    pltpu.sync_copy(zvec.at[pl.ds(sid * RPSD, RPSD)],
                    den_sh.at[pl.ds(sid * RPSD, RPSD)])
    plsc.subcore_barrier()

    iota16 = lax.iota(i32, 16)
    base = wid * NCHUNKL
    bufs = ((pack0, d0, e0, q0, f0, r0, m0, sg0, ss0),
            (pack1, d1, e1, q1, f1, r1, m1, sg1, ss1))

    def fire_gathers(b, c):
        pack, dv, ev, qv, fv, rv, mv, sg, _ = bufs[b]
        pltpu.sync_copy(packl.at[pl.ds(c * (3 * CHL), 3 * CHL)], pack)
        pltpu.sync_copy(dstp.at[pl.ds(c * CHL, CHL)], dv)
        pltpu.async_copy(eflat.at[pack.at[pl.ds(2 * CHL, CHL)]], ev, sg)
        pltpu.async_copy(qflat.at[pack.at[pl.ds(2 * CHL, CHL)]], qv, sg)
        pltpu.async_copy(fmat.at[pack.at[pl.ds(0, CHL)]], fv, sg)
        pltpu.async_copy(rn.at[pack.at[pl.ds(CHL, CHL)]], rv, sg)

    def wait_gathers(b):
        pack, dv, ev, qv, fv, rv, mv, sg, _ = bufs[b]
        pltpu.make_async_copy(eflat.at[pack.at[pl.ds(2 * CHL, CHL)]], ev, sg).wait()
        pltpu.make_async_copy(qflat.at[pack.at[pl.ds(2 * CHL, CHL)]], qv, sg).wait()
        pltpu.make_async_copy(fmat.at[pack.at[pl.ds(0, CHL)]], fv, sg).wait()
        pltpu.make_async_copy(rn.at[pack.at[pl.ds(CHL, CHL)]], rv, sg).wait()

    def rowloop(b):
        pack, dv, ev, qv, fv, rv, mv, sg, _ = bufs[b]

        def row_body(i, rc):
            ridx = lax.broadcast(i, (16,))
            p16 = plsc.load_gather(ev, [ridx])
            q16 = plsc.load_gather(qv, [ridx])
            for j in range(D // 16):
                cidx = iota16 + (j * 16)
                fvv = plsc.load_gather(fv, [ridx, cidx])
                rvv = plsc.load_gather(rv, [ridx, cidx])
                plsc.store_scatter(mv, [ridx, cidx],
                                   p16 * fvv - q16 * rvv)
            return rc

        lax.fori_loop(0, CHL, row_body, 0)

    def fire_scatters(b):
        pack, dv, ev, qv, fv, rv, mv, _, ss = bufs[b]
        d1 = pltpu.async_copy(ev, den_sh.at[dv], ss, add=True)
        d2 = pltpu.async_copy(mv, acc_sh.at[dv], ss, add=True)
        return d1, d2

    fire_gathers(0, base)
    fire_gathers(1, base + 1)

    def pair(i, carry):
        c0 = 2 * i
        wait_gathers(0)
        rowloop(0)
        s0 = fire_scatters(0)
        wait_gathers(1)
        rowloop(1)
        s1 = fire_scatters(1)
        s0[0].wait()
        s0[1].wait()
        fire_gathers(0, base + jnp.minimum(c0 + 2, NCHUNKL - 1))
        s1[0].wait()
        s1[1].wait()
        fire_gathers(1, base + jnp.minimum(c0 + 3, NCHUNKL - 1))
        return carry

    lax.fori_loop(0, NCHUNKL // 2, pair, 0)
    wait_gathers(0)
    wait_gathers(1)
    plsc.subcore_barrier()
    pltpu.sync_copy(acc_sh.at[pl.ds(sid * RPS, RPS)],
                    acc_out.at[cid, pl.ds(sid * RPS, RPS)])
    pltpu.sync_copy(den_sh.at[pl.ds(sid * RPSD, RPSD)],
                    den_out.at[cid, pl.ds(sid * RPSD, RPSD)])


_layer_pass = pl.kernel(
    _layer_body,
    out_type=[jax.ShapeDtypeStruct((NC, NP, D), f32),
              jax.ShapeDtypeStruct((NC, NP), f32)],
    mesh=_MESH,
    compiler_params=_SC_PARAMS,
    scratch_types=[
        pltpu.VMEM((3 * CHL,), i32),
        pltpu.VMEM((3 * CHL,), i32),
        pltpu.VMEM((CHL,), i32),
        pltpu.VMEM((CHL,), i32),
        pltpu.VMEM((CHL,), f32),
        pltpu.VMEM((CHL,), f32),
        pltpu.VMEM((CHL,), f32),
        pltpu.VMEM((CHL,), f32),
        pltpu.VMEM((CHL, D), f32),
        pltpu.VMEM((CHL, D), f32),
        pltpu.VMEM((CHL, D), f32),
        pltpu.VMEM((CHL, D), f32),
        pltpu.VMEM((CHL, D), f32),
        pltpu.VMEM((CHL, D), f32),
        pltpu.VMEM_SHARED((NPA, D), f32),
        pltpu.VMEM_SHARED((NP,), f32),
        pltpu.SemaphoreType.DMA,
        pltpu.SemaphoreType.DMA,
        pltpu.SemaphoreType.DMA,
        pltpu.SemaphoreType.DMA,
    ],
)


def _prep_rel_kernel(r_ref, k1_ref, k2_ref, rn_ref, b_ref, c_ref):
    r = r_ref[...]
    nrm = jnp.sqrt(jnp.sum(r * r, axis=1, keepdims=True))
    rn = r / jnp.maximum(nrm, 1e-12)
    rn_ref[...] = rn
    dn = (((1,), (1,)), ((), ()))
    b_ref[...] = lax.dot_general(k1_ref[...], rn, dn,
                                 preferred_element_type=f32)
    c_ref[...] = lax.dot_general(k2_ref[...], rn, dn,
                                 preferred_element_type=f32)


def _prep_rel(rel_pad, k1s, k2s):
    return pl.pallas_call(
        _prep_rel_kernel,
        out_shape=[jax.ShapeDtypeStruct((RP, D), f32),
                   jax.ShapeDtypeStruct((4, RP), f32),
                   jax.ShapeDtypeStruct((4, RP), f32)],
    )(rel_pad, k1s, k2s)


_BLK = 512
_DN = (((1,), (1,)), ((), ()))


def _prep_layer_kernel(use_deg, slot,
                       acc_ref, den_ref, rn_ref, k1_ref, b_ref, c_ref,
                       f_ref, e_ref, q_ref):
    if use_deg:
        a = acc_ref[slot]
        invd = 1.0 / jnp.maximum(den_ref[0], 1.0)
    else:
        a = acc_ref[0] + acc_ref[1]
        dn = den_ref[0] + den_ref[1]
        invd = jnp.where(dn > 0.0, 1.0 / dn, 0.0)
    fmat = jnp.tanh(a * invd[:, None])
    f_ref[...] = fmat
    g = lax.dot_general(fmat, rn_ref[...], _DN, preferred_element_type=f32)
    an = lax.dot_general(fmat, k1_ref[...], _DN, preferred_element_type=f32)
    e = jnp.exp(an + c_ref[...] - 2.0 * g * b_ref[...])
    e_ref[...] = e
    q_ref[...] = 2.0 * e * g


def _prep_layer(acc, den, rn, k1, b, c, use_deg, slot):
    grid = NP // _BLK
    outs = [jax.ShapeDtypeStruct((NP, D), f32),
            jax.ShapeDtypeStruct((NP, RP), f32),
            jax.ShapeDtypeStruct((NP, RP), f32)]
    out_specs = [pl.BlockSpec((_BLK, D), lambda i: (i, 0)),
                 pl.BlockSpec((_BLK, RP), lambda i: (i, 0)),
                 pl.BlockSpec((_BLK, RP), lambda i: (i, 0))]
    return pl.pallas_call(
        functools.partial(_prep_layer_kernel, use_deg, slot),
        grid=(grid,),
        in_specs=[pl.BlockSpec((NC, _BLK, D), lambda i: (0, i, 0)),
                  pl.BlockSpec((NC, _BLK), lambda i: (0, i)),
                  pl.BlockSpec((RP, D), lambda i: (0, 0)),
                  pl.BlockSpec((1, D), lambda i: (0, 0)),
                  pl.BlockSpec((1, RP), lambda i: (0, 0)),
                  pl.BlockSpec((1, RP), lambda i: (0, 0))],
        out_specs=out_specs,
        out_shape=outs,
    )(acc, den, rn, k1, b, c)


def _final_kernel(acc_ref, den_ref, f_ref):
    a = acc_ref[0] + acc_ref[1]
    dn = den_ref[0] + den_ref[1]
    invd = jnp.where(dn > 0.0, 1.0 / dn, 0.0)
    f_ref[...] = jnp.tanh(a * invd[:, None])


def _finalize(acc, den):
    return pl.pallas_call(
        _final_kernel,
        grid=(NP // _BLK,),
        in_specs=[pl.BlockSpec((NC, _BLK, D), lambda i: (0, i, 0)),
                  pl.BlockSpec((NC, _BLK), lambda i: (0, i))],
        out_specs=pl.BlockSpec((_BLK, D), lambda i: (i, 0)),
        out_shape=jax.ShapeDtypeStruct((NP, D), f32),
    )(acc, den)


def kernel(edge_index, edge_rel, ent_emb, rel_emb, attn_e, attn_r):
    src = edge_index[0]
    dst = edge_index[1]
    pad = EP - E
    srcp = jnp.concatenate([src, jnp.zeros((pad,), i32)])
    dstp = jnp.concatenate([dst, jnp.full((pad,), DUMMY, i32)])
    relp = jnp.concatenate([edge_rel, jnp.zeros((pad,), i32)])
    order = jnp.argsort(dstp)
    srcp = srcp[order]
    dstp = dstp[order]
    relp = relp[order]
    gidxp = srcp * RP + relp
    epack0 = (jnp.stack([srcp, dstp, relp, gidxp])
              .reshape(4, EP // CH0, CH0).transpose(1, 0, 2))
    packl = (jnp.stack([srcp, relp, gidxp])
             .reshape(3, EP // CHL, CHL).transpose(1, 0, 2).reshape(-1))
    rel_pad = jnp.pad(rel_emb, ((0, RP - NREL), (0, 0)))
    ctab0 = jnp.concatenate([ent_emb, rel_emb])

    attn = [attn_e, attn_r]
    k1s = jnp.stack([attn[enc][l, 128:256, 0]
                     for enc in range(2) for l in range(2)])
    k2s = jnp.stack([attn[enc][l, 256:384, 0]
                     for enc in range(2) for l in range(2)])
    rn, btab, ctab = _prep_rel(rel_pad, k1s, k2s)

    zrows = jnp.zeros((NPA, D), f32)
    zvec = jnp.zeros((NP,), f32)

    acc0, den0 = _pass0(ctab0, epack0, zrows, zvec)

    outs = []
    for enc in range(2):
        acc, den = acc0, den0
        use_deg = True
        for l in range(2):
            ki = enc * 2 + l
            fmat, emat, qmat = _prep_layer(
                acc, den, rn, k1s[ki][None], btab[ki][None], ctab[ki][None],
                use_deg, enc)
            outs.append(fmat)
            acc, den = _layer_pass(
                packl, dstp, emat.reshape(NP * RP), qmat.reshape(NP * RP),
                fmat, rn, zrows, zvec)
            use_deg = False
        outs.append(_finalize(acc, den))

    return jnp.concatenate(outs, axis=1)[:N]


# speedup vs baseline: 1.4130x; 1.4130x over previous
import functools

import jax
import jax.numpy as jnp
from jax import lax
from jax.experimental import pallas as pl
from jax.experimental.pallas import tpu as pltpu
from jax.experimental.pallas import tpu_sc as plsc

N = 10000
NP = 10240
E = 320000
EP = 327680
NREL = 1000
RP = 1024
D = 128
NC = 2
NS = 16
NW = NC * NS
EPT = EP // NW
CH0 = 128
NCHUNK0 = EP // NS // CH0
CHL = 64
NCHUNKL = EPT // CHL
RPS = NP // NS
DUMMY = N + 1

_MESH = plsc.VectorSubcoreMesh(core_axis_name="c", subcore_axis_name="s")
_SC_PARAMS = pltpu.CompilerParams(needs_layout_passes=False)
f32 = jnp.float32
i32 = jnp.int32


def _pass0_body(ctab, epack, zrows, zvec,
                acc_out, den_out,
                pack0, pack1, cidx0, cidx1, ones_v, rows0, rows1,
                acc_sh, den_sh, si0, si1, sg0, sg1, ss0, ss1):
    cid = lax.axis_index("c")
    sid = lax.axis_index("s")
    pltpu.sync_copy(zrows.at[pl.ds(sid * RPS, RPS)],
                    acc_sh.at[pl.ds(sid * RPS, RPS)])
    pltpu.sync_copy(zvec.at[pl.ds(sid * RPS, RPS)],
                    den_sh.at[pl.ds(sid * RPS, RPS)])
    for k in range(CH0 // 16):
        ones_v[pl.ds(k * 16, 16)] = jnp.ones((16,), f32)
    plsc.subcore_barrier()

    is_ent = lax.broadcast(cid == 0, (16,))
    base = sid * NCHUNK0
    bufs = ((pack0, cidx0, rows0, si0, sg0, ss0),
            (pack1, cidx1, rows1, si1, sg1, ss1))

    def fire_pack(b, c):
        pack, cv, rows, si, sg, _ = bufs[b]
        pltpu.async_copy(epack.at[c], pack, si)

    def fire_gathers(b):
        pack, cv, rows, si, sg, _ = bufs[b]
        pltpu.make_async_copy(epack.at[0], pack, si).wait()
        for k in range(CH0 // 16):
            sl = pl.ds(k * 16, 16)
            cv[sl] = jnp.where(is_ent, pack[0, sl], pack[2, sl] + N)
        pltpu.async_copy(ctab.at[cv], rows, sg)

    def wait_gathers(b):
        pack, cv, rows, si, sg, _ = bufs[b]
        pltpu.make_async_copy(ctab.at[cv], rows, sg).wait()

    def fire_scatters(b):
        pack, cv, rows, si, sg, ss = bufs[b]
        d1 = pltpu.async_copy(rows, acc_sh.at[pack.at[1]], ss, add=True)
        d2 = pltpu.async_copy(ones_v, den_sh.at[pack.at[1]], ss, add=True)
        return d1, d2

    fire_pack(0, base)
    fire_pack(1, base + 1)
    fire_gathers(0)
    fire_gathers(1)

    def pair(i, carry):
        c0 = 2 * i
        wait_gathers(0)
        s0 = fire_scatters(0)
        wait_gathers(1)
        s1 = fire_scatters(1)
        s0[0].wait()
        s0[1].wait()
        fire_pack(0, base + jnp.minimum(c0 + 2, NCHUNK0 - 1))
        s1[0].wait()
        s1[1].wait()
        fire_pack(1, base + jnp.minimum(c0 + 3, NCHUNK0 - 1))
        fire_gathers(0)
        fire_gathers(1)
        return carry

    lax.fori_loop(0, NCHUNK0 // 2, pair, 0)
    wait_gathers(0)
    wait_gathers(1)
    plsc.subcore_barrier()
    pltpu.sync_copy(acc_sh.at[pl.ds(sid * RPS, RPS)],
                    acc_out.at[cid, pl.ds(sid * RPS, RPS)])
    pltpu.sync_copy(den_sh.at[pl.ds(sid * RPS, RPS)],
                    den_out.at[cid, pl.ds(sid * RPS, RPS)])


_pass0 = pl.kernel(
    _pass0_body,
    out_type=[jax.ShapeDtypeStruct((NC, NP, D), f32),
              jax.ShapeDtypeStruct((NC, NP), f32)],
    mesh=_MESH,
    compiler_params=_SC_PARAMS,
    scratch_types=[
        pltpu.VMEM((4, CH0), i32),
        pltpu.VMEM((4, CH0), i32),
        pltpu.VMEM((CH0,), i32),
        pltpu.VMEM((CH0,), i32),
        pltpu.VMEM((CH0,), f32),
        pltpu.VMEM((CH0, D), f32),
        pltpu.VMEM((CH0, D), f32),
        pltpu.VMEM_SHARED((NP, D), f32),
        pltpu.VMEM_SHARED((NP,), f32),
        pltpu.SemaphoreType.DMA,
        pltpu.SemaphoreType.DMA,
        pltpu.SemaphoreType.DMA,
        pltpu.SemaphoreType.DMA,
        pltpu.SemaphoreType.DMA,
        pltpu.SemaphoreType.DMA,
    ],
)


def _layer_body(epack, eflat, qflat, fmat, rn, zrows, zvec,
                acc_out, den_out,
                pack0, pack1, e0, e1, q0, q1, f0, f1, r0, r1,
                acc_sh, den_sh, si0, si1, sg0, sg1, ss0, ss1):
    cid = lax.axis_index("c")
    sid = lax.axis_index("s")
    wid = sid * NC + cid
    pltpu.sync_copy(zrows.at[pl.ds(sid * RPS, RPS)],
                    acc_sh.at[pl.ds(sid * RPS, RPS)])
    pltpu.sync_copy(zvec.at[pl.ds(sid * RPS, RPS)],
                    den_sh.at[pl.ds(sid * RPS, RPS)])
    plsc.subcore_barrier()

    iota16 = lax.iota(i32, 16)
    base = wid * NCHUNKL
    bufs = ((pack0, e0, q0, f0, r0, si0, sg0, ss0),
            (pack1, e1, q1, f1, r1, si1, sg1, ss1))

    def fire_pack(b, c):
        pack, ev, qv, fv, rv, si, sg, _ = bufs[b]
        pltpu.async_copy(epack.at[c], pack, si)

    def fire_gathers(b):
        pack, ev, qv, fv, rv, si, sg, _ = bufs[b]
        pltpu.make_async_copy(epack.at[0], pack, si).wait()
        pltpu.async_copy(eflat.at[pack.at[3]], ev, sg)
        pltpu.async_copy(qflat.at[pack.at[3]], qv, sg)
        pltpu.async_copy(fmat.at[pack.at[0]], fv, sg)
        pltpu.async_copy(rn.at[pack.at[2]], rv, sg)

    def wait_gathers(b):
        pack, ev, qv, fv, rv, si, sg, _ = bufs[b]
        pltpu.make_async_copy(eflat.at[pack.at[3]], ev, sg).wait()
        pltpu.make_async_copy(qflat.at[pack.at[3]], qv, sg).wait()
        pltpu.make_async_copy(fmat.at[pack.at[0]], fv, sg).wait()
        pltpu.make_async_copy(rn.at[pack.at[2]], rv, sg).wait()

    def rowloop(b):
        pack, ev, qv, fv, rv, si, sg, _ = bufs[b]

        def row_body(i, rc):
            ridx = lax.broadcast(i, (16,))
            p16 = plsc.load_gather(ev, [ridx])
            q16 = plsc.load_gather(qv, [ridx])
            for j in range(D // 16):
                cidx = iota16 + (j * 16)
                fvv = plsc.load_gather(fv, [ridx, cidx])
                rvv = plsc.load_gather(rv, [ridx, cidx])
                plsc.store_scatter(fv, [ridx, cidx],
                                   p16 * fvv - q16 * rvv)
            return rc

        lax.fori_loop(0, CHL, row_body, 0)

    def fire_scatters(b):
        pack, ev, qv, fv, rv, si, _, ss = bufs[b]
        d1 = pltpu.async_copy(ev, den_sh.at[pack.at[1]], ss, add=True)
        d2 = pltpu.async_copy(fv, acc_sh.at[pack.at[1]], ss, add=True)
        return d1, d2

    fire_pack(0, base)
    fire_pack(1, base + 1)
    fire_gathers(0)
    fire_gathers(1)

    def pair(i, carry):
        c0 = 2 * i
        wait_gathers(0)
        fire_pack(0, base + jnp.minimum(c0 + 2, NCHUNKL - 1))
        rowloop(0)
        s0 = fire_scatters(0)
        wait_gathers(1)
        fire_pack(1, base + jnp.minimum(c0 + 3, NCHUNKL - 1))
        rowloop(1)
        s1 = fire_scatters(1)
        s0[0].wait()
        s0[1].wait()
        fire_gathers(0)
        s1[0].wait()
        s1[1].wait()
        fire_gathers(1)
        return carry

    lax.fori_loop(0, NCHUNKL // 2, pair, 0)
    wait_gathers(0)
    wait_gathers(1)
    plsc.subcore_barrier()
    pltpu.sync_copy(acc_sh.at[pl.ds(sid * RPS, RPS)],
                    acc_out.at[cid, pl.ds(sid * RPS, RPS)])
    pltpu.sync_copy(den_sh.at[pl.ds(sid * RPS, RPS)],
                    den_out.at[cid, pl.ds(sid * RPS, RPS)])


_layer_pass = pl.kernel(
    _layer_body,
    out_type=[jax.ShapeDtypeStruct((NC, NP, D), f32),
              jax.ShapeDtypeStruct((NC, NP), f32)],
    mesh=_MESH,
    compiler_params=_SC_PARAMS,
    scratch_types=[
        pltpu.VMEM((4, CHL), i32),
        pltpu.VMEM((4, CHL), i32),
        pltpu.VMEM((CHL,), f32),
        pltpu.VMEM((CHL,), f32),
        pltpu.VMEM((CHL,), f32),
        pltpu.VMEM((CHL,), f32),
        pltpu.VMEM((CHL, D), f32),
        pltpu.VMEM((CHL, D), f32),
        pltpu.VMEM((CHL, D), f32),
        pltpu.VMEM((CHL, D), f32),
        pltpu.VMEM_SHARED((NP, D), f32),
        pltpu.VMEM_SHARED((NP,), f32),
        pltpu.SemaphoreType.DMA,
        pltpu.SemaphoreType.DMA,
        pltpu.SemaphoreType.DMA,
        pltpu.SemaphoreType.DMA,
        pltpu.SemaphoreType.DMA,
        pltpu.SemaphoreType.DMA,
    ],
)


def _prep_rel_kernel(r_ref, k1_ref, k2_ref, rn_ref, b_ref, c_ref):
    r = r_ref[...]
    nrm = jnp.sqrt(jnp.sum(r * r, axis=1, keepdims=True))
    rn = r / jnp.maximum(nrm, 1e-12)
    rn_ref[...] = rn
    dn = (((1,), (1,)), ((), ()))
    b_ref[...] = lax.dot_general(k1_ref[...], rn, dn,
                                 preferred_element_type=f32)
    c_ref[...] = lax.dot_general(k2_ref[...], rn, dn,
                                 preferred_element_type=f32)


def _prep_rel(rel_pad, k1s, k2s):
    return pl.pallas_call(
        _prep_rel_kernel,
        out_shape=[jax.ShapeDtypeStruct((RP, D), f32),
                   jax.ShapeDtypeStruct((4, RP), f32),
                   jax.ShapeDtypeStruct((4, RP), f32)],
    )(rel_pad, k1s, k2s)


_BLK = 512
_DN = (((1,), (1,)), ((), ()))


def _prep_layer_kernel(use_deg, slot,
                       acc_ref, den_ref, rn_ref, k1_ref, b_ref, c_ref,
                       f_ref, e_ref, q_ref):
    if use_deg:
        a = acc_ref[slot]
        invd = 1.0 / jnp.maximum(den_ref[0], 1.0)
    else:
        a = acc_ref[0] + acc_ref[1]
        dn = den_ref[0] + den_ref[1]
        invd = jnp.where(dn > 0.0, 1.0 / dn, 0.0)
    fmat = jnp.tanh(a * invd[:, None])
    f_ref[...] = fmat
    g = lax.dot_general(fmat, rn_ref[...], _DN, preferred_element_type=f32)
    an = lax.dot_general(fmat, k1_ref[...], _DN, preferred_element_type=f32)
    e = jnp.exp(an + c_ref[...] - 2.0 * g * b_ref[...])
    e_ref[...] = e
    q_ref[...] = 2.0 * e * g


def _prep_layer(acc, den, rn, k1, b, c, use_deg, slot):
    grid = NP // _BLK
    outs = [jax.ShapeDtypeStruct((NP, D), f32),
            jax.ShapeDtypeStruct((NP, RP), f32),
            jax.ShapeDtypeStruct((NP, RP), f32)]
    out_specs = [pl.BlockSpec((_BLK, D), lambda i: (i, 0)),
                 pl.BlockSpec((_BLK, RP), lambda i: (i, 0)),
                 pl.BlockSpec((_BLK, RP), lambda i: (i, 0))]
    return pl.pallas_call(
        functools.partial(_prep_layer_kernel, use_deg, slot),
        grid=(grid,),
        in_specs=[pl.BlockSpec((NC, _BLK, D), lambda i: (0, i, 0)),
                  pl.BlockSpec((NC, _BLK), lambda i: (0, i)),
                  pl.BlockSpec((RP, D), lambda i: (0, 0)),
                  pl.BlockSpec((1, D), lambda i: (0, 0)),
                  pl.BlockSpec((1, RP), lambda i: (0, 0)),
                  pl.BlockSpec((1, RP), lambda i: (0, 0))],
        out_specs=out_specs,
        out_shape=outs,
    )(acc, den, rn, k1, b, c)


def _final_kernel(acc_ref, den_ref, f_ref):
    a = acc_ref[0] + acc_ref[1]
    dn = den_ref[0] + den_ref[1]
    invd = jnp.where(dn > 0.0, 1.0 / dn, 0.0)
    f_ref[...] = jnp.tanh(a * invd[:, None])


def _finalize(acc, den):
    return pl.pallas_call(
        _final_kernel,
        grid=(NP // _BLK,),
        in_specs=[pl.BlockSpec((NC, _BLK, D), lambda i: (0, i, 0)),
                  pl.BlockSpec((NC, _BLK), lambda i: (0, i))],
        out_specs=pl.BlockSpec((_BLK, D), lambda i: (i, 0)),
        out_shape=jax.ShapeDtypeStruct((NP, D), f32),
    )(acc, den)


def kernel(edge_index, edge_rel, ent_emb, rel_emb, attn_e, attn_r):
    src = edge_index[0]
    dst = edge_index[1]
    pad = EP - E
    srcp = jnp.concatenate([src, jnp.zeros((pad,), i32)])
    dstp = jnp.concatenate([dst, jnp.full((pad,), DUMMY, i32)])
    relp = jnp.concatenate([edge_rel, jnp.zeros((pad,), i32)])
    gidxp = srcp * RP + relp
    stacked = jnp.stack([srcp, dstp, relp, gidxp])
    epack0 = stacked.reshape(4, EP // CH0, CH0).transpose(1, 0, 2)
    epackl = stacked.reshape(4, EP // CHL, CHL).transpose(1, 0, 2)
    rel_pad = jnp.pad(rel_emb, ((0, RP - NREL), (0, 0)))
    ctab0 = jnp.concatenate([ent_emb, rel_emb])

    attn = [attn_e, attn_r]
    k1s = jnp.stack([attn[enc][l, 128:256, 0]
                     for enc in range(2) for l in range(2)])
    k2s = jnp.stack([attn[enc][l, 256:384, 0]
                     for enc in range(2) for l in range(2)])
    rn, btab, ctab = _prep_rel(rel_pad, k1s, k2s)

    zrows = jnp.zeros((NP, D), f32)
    zvec = jnp.zeros((NP,), f32)

    acc0, den0 = _pass0(ctab0, epack0, zrows, zvec)

    outs = []
    for enc in range(2):
        acc, den = acc0, den0
        use_deg = True
        for l in range(2):
            ki = enc * 2 + l
            fmat, emat, qmat = _prep_layer(
                acc, den, rn, k1s[ki][None], btab[ki][None], ctab[ki][None],
                use_deg, enc)
            outs.append(fmat)
            acc, den = _layer_pass(
                epackl, emat.reshape(NP * RP), qmat.reshape(NP * RP),
                fmat, rn, zrows, zvec)
            use_deg = False
        outs.append(_finalize(acc, den))

    return jnp.concatenate(outs, axis=1)[:N]


# speedup vs baseline: 1.4139x; 1.0006x over previous
import functools

import jax
import jax.numpy as jnp
from jax import lax
from jax.experimental import pallas as pl
from jax.experimental.pallas import tpu as pltpu
from jax.experimental.pallas import tpu_sc as plsc

N = 10000
NP = 10240
E = 320000
EP = 327680
NREL = 1000
RP = 1024
D = 128
NC = 2
NS = 16
NW = NC * NS
EPT = EP // NW
CH0 = 128
NCHUNK0 = EP // NS // CH0
CHL = 64
NCHUNKL = EPT // CHL
RPS = NP // NS
DUMMY = N + 1

_MESH = plsc.VectorSubcoreMesh(core_axis_name="c", subcore_axis_name="s")
_SC_PARAMS = pltpu.CompilerParams(needs_layout_passes=False)
f32 = jnp.float32
i32 = jnp.int32


def _pass0_body(ctab, epack, zrows, zvec,
                acc_out, den_out,
                pack0, pack1, cidx0, cidx1, ones_v, rows0, rows1,
                acc_sh, den_sh, si0, si1, sg0, sg1, ss0, ss1):
    cid = lax.axis_index("c")
    sid = lax.axis_index("s")
    pltpu.sync_copy(zrows.at[pl.ds(sid * RPS, RPS)],
                    acc_sh.at[pl.ds(sid * RPS, RPS)])
    pltpu.sync_copy(zvec.at[pl.ds(sid * RPS, RPS)],
                    den_sh.at[pl.ds(sid * RPS, RPS)])
    for k in range(CH0 // 16):
        ones_v[pl.ds(k * 16, 16)] = jnp.ones((16,), f32)
    plsc.subcore_barrier()

    is_ent = lax.broadcast(cid == 0, (16,))
    base = sid * NCHUNK0
    bufs = ((pack0, cidx0, rows0, si0, sg0, ss0),
            (pack1, cidx1, rows1, si1, sg1, ss1))

    def fire_pack(b, c):
        pack, cv, rows, si, sg, _ = bufs[b]
        pltpu.async_copy(epack.at[c], pack, si)

    def fire_gathers(b):
        pack, cv, rows, si, sg, _ = bufs[b]
        pltpu.make_async_copy(epack.at[0], pack, si).wait()
        for k in range(CH0 // 16):
            sl = pl.ds(k * 16, 16)
            cv[sl] = jnp.where(is_ent, pack[0, sl], pack[2, sl] + N)
        pltpu.async_copy(ctab.at[cv], rows, sg)

    def wait_gathers(b):
        pack, cv, rows, si, sg, _ = bufs[b]
        pltpu.make_async_copy(ctab.at[cv], rows, sg).wait()

    def fire_scatters(b):
        pack, cv, rows, si, sg, ss = bufs[b]
        d1 = pltpu.async_copy(rows, acc_sh.at[pack.at[1]], ss, add=True)
        d2 = pltpu.async_copy(ones_v, den_sh.at[pack.at[1]], ss, add=True)
        return d1, d2

    fire_pack(0, base)
    fire_pack(1, base + 1)
    fire_gathers(0)
    fire_gathers(1)

    def pair(i, carry):
        c0 = 2 * i
        wait_gathers(0)
        s0 = fire_scatters(0)
        wait_gathers(1)
        s1 = fire_scatters(1)
        s0[0].wait()
        s0[1].wait()
        fire_pack(0, base + jnp.minimum(c0 + 2, NCHUNK0 - 1))
        s1[0].wait()
        s1[1].wait()
        fire_pack(1, base + jnp.minimum(c0 + 3, NCHUNK0 - 1))
        fire_gathers(0)
        fire_gathers(1)
        return carry

    lax.fori_loop(0, NCHUNK0 // 2, pair, 0)
    wait_gathers(0)
    wait_gathers(1)
    plsc.subcore_barrier()
    pltpu.sync_copy(acc_sh.at[pl.ds(sid * RPS, RPS)],
                    acc_out.at[cid, pl.ds(sid * RPS, RPS)])
    pltpu.sync_copy(den_sh.at[pl.ds(sid * RPS, RPS)],
                    den_out.at[cid, pl.ds(sid * RPS, RPS)])


_pass0 = pl.kernel(
    _pass0_body,
    out_type=[jax.ShapeDtypeStruct((NC, NP, D), f32),
              jax.ShapeDtypeStruct((NC, NP), f32)],
    mesh=_MESH,
    compiler_params=_SC_PARAMS,
    scratch_types=[
        pltpu.VMEM((4, CH0), i32),
        pltpu.VMEM((4, CH0), i32),
        pltpu.VMEM((CH0,), i32),
        pltpu.VMEM((CH0,), i32),
        pltpu.VMEM((CH0,), f32),
        pltpu.VMEM((CH0, D), f32),
        pltpu.VMEM((CH0, D), f32),
        pltpu.VMEM_SHARED((NP, D), f32),
        pltpu.VMEM_SHARED((NP,), f32),
        pltpu.SemaphoreType.DMA,
        pltpu.SemaphoreType.DMA,
        pltpu.SemaphoreType.DMA,
        pltpu.SemaphoreType.DMA,
        pltpu.SemaphoreType.DMA,
        pltpu.SemaphoreType.DMA,
    ],
)


def _layer_body(epack, eflat, qflat, fmat, rn, zrows, zvec,
                acc_out, den_out,
                pack0, pack1, dv0, dv1, e0, e1, q0, q1, f0, f1, r0, r1,
                acc_sh, den_sh, si0, si1, sg0, sg1, ss0, ss1):
    cid = lax.axis_index("c")
    sid = lax.axis_index("s")
    wid = sid * NC + cid
    pltpu.sync_copy(zrows.at[pl.ds(sid * RPS, RPS)],
                    acc_sh.at[pl.ds(sid * RPS, RPS)])
    pltpu.sync_copy(zvec.at[pl.ds(sid * RPS, RPS)],
                    den_sh.at[pl.ds(sid * RPS, RPS)])
    plsc.subcore_barrier()

    iota16 = lax.iota(i32, 16)
    base = wid * NCHUNKL
    bufs = ((pack0, dv0, e0, q0, f0, r0, si0, sg0, ss0),
            (pack1, dv1, e1, q1, f1, r1, si1, sg1, ss1))

    def fire_pack(b, c):
        pack, dv, ev, qv, fv, rv, si, sg, _ = bufs[b]
        pltpu.async_copy(epack.at[c], pack, si)

    def fire_gathers(b):
        pack, dv, ev, qv, fv, rv, si, sg, _ = bufs[b]
        pltpu.make_async_copy(epack.at[0], pack, si).wait()
        pltpu.async_copy(eflat.at[pack.at[3]], ev, sg)
        pltpu.async_copy(qflat.at[pack.at[3]], qv, sg)
        pltpu.async_copy(fmat.at[pack.at[0]], fv, sg)
        pltpu.async_copy(rn.at[pack.at[2]], rv, sg)

    def wait_gathers(b):
        pack, dv, ev, qv, fv, rv, si, sg, _ = bufs[b]
        pltpu.make_async_copy(eflat.at[pack.at[3]], ev, sg).wait()
        pltpu.make_async_copy(qflat.at[pack.at[3]], qv, sg).wait()
        pltpu.make_async_copy(fmat.at[pack.at[0]], fv, sg).wait()
        pltpu.make_async_copy(rn.at[pack.at[2]], rv, sg).wait()
        for k in range(CHL // 16):
            sl = pl.ds(k * 16, 16)
            dv[sl] = pack[1, sl]

    def rowloop(b):
        pack, dv, ev, qv, fv, rv, si, sg, _ = bufs[b]

        def row_body(i, rc):
            ridx = lax.broadcast(i, (16,))
            p16 = plsc.load_gather(ev, [ridx])
            q16 = plsc.load_gather(qv, [ridx])
            for j in range(D // 16):
                cidx = iota16 + (j * 16)
                fvv = plsc.load_gather(fv, [ridx, cidx])
                rvv = plsc.load_gather(rv, [ridx, cidx])
                plsc.store_scatter(fv, [ridx, cidx],
                                   p16 * fvv - q16 * rvv)
            return rc

        lax.fori_loop(0, CHL, row_body, 0)

    def fire_scatters(b):
        pack, dv, ev, qv, fv, rv, si, _, ss = bufs[b]
        d1 = pltpu.async_copy(ev, den_sh.at[dv], ss, add=True)
        d2 = pltpu.async_copy(fv, acc_sh.at[dv], ss, add=True)
        return d1, d2

    fire_pack(0, base)
    fire_pack(1, base + 1)
    fire_gathers(0)
    fire_gathers(1)

    def pair(i, carry):
        c0 = 2 * i
        wait_gathers(0)
        fire_pack(0, base + jnp.minimum(c0 + 2, NCHUNKL - 1))
        rowloop(0)
        s0 = fire_scatters(0)
        wait_gathers(1)
        fire_pack(1, base + jnp.minimum(c0 + 3, NCHUNKL - 1))
        rowloop(1)
        s1 = fire_scatters(1)
        s0[0].wait()
        s0[1].wait()
        fire_gathers(0)
        s1[0].wait()
        s1[1].wait()
        fire_gathers(1)
        return carry

    lax.fori_loop(0, NCHUNKL // 2, pair, 0)
    wait_gathers(0)
    wait_gathers(1)
    plsc.subcore_barrier()
    pltpu.sync_copy(acc_sh.at[pl.ds(sid * RPS, RPS)],
                    acc_out.at[cid, pl.ds(sid * RPS, RPS)])
    pltpu.sync_copy(den_sh.at[pl.ds(sid * RPS, RPS)],
                    den_out.at[cid, pl.ds(sid * RPS, RPS)])


_layer_pass = pl.kernel(
    _layer_body,
    out_type=[jax.ShapeDtypeStruct((NC, NP, D), f32),
              jax.ShapeDtypeStruct((NC, NP), f32)],
    mesh=_MESH,
    compiler_params=_SC_PARAMS,
    scratch_types=[
        pltpu.VMEM((4, CHL), i32),
        pltpu.VMEM((4, CHL), i32),
        pltpu.VMEM((CHL,), i32),
        pltpu.VMEM((CHL,), i32),
        pltpu.VMEM((CHL,), f32),
        pltpu.VMEM((CHL,), f32),
        pltpu.VMEM((CHL,), f32),
        pltpu.VMEM((CHL,), f32),
        pltpu.VMEM((CHL, D), f32),
        pltpu.VMEM((CHL, D), f32),
        pltpu.VMEM((CHL, D), f32),
        pltpu.VMEM((CHL, D), f32),
        pltpu.VMEM_SHARED((NP, D), f32),
        pltpu.VMEM_SHARED((NP,), f32),
        pltpu.SemaphoreType.DMA,
        pltpu.SemaphoreType.DMA,
        pltpu.SemaphoreType.DMA,
        pltpu.SemaphoreType.DMA,
        pltpu.SemaphoreType.DMA,
        pltpu.SemaphoreType.DMA,
    ],
)


def _prep_rel_kernel(r_ref, k1_ref, k2_ref, rn_ref, b_ref, c_ref):
    r = r_ref[...]
    nrm = jnp.sqrt(jnp.sum(r * r, axis=1, keepdims=True))
    rn = r / jnp.maximum(nrm, 1e-12)
    rn_ref[...] = rn
    dn = (((1,), (1,)), ((), ()))
    b_ref[...] = lax.dot_general(k1_ref[...], rn, dn,
                                 preferred_element_type=f32)
    c_ref[...] = lax.dot_general(k2_ref[...], rn, dn,
                                 preferred_element_type=f32)


def _prep_rel(rel_pad, k1s, k2s):
    return pl.pallas_call(
        _prep_rel_kernel,
        out_shape=[jax.ShapeDtypeStruct((RP, D), f32),
                   jax.ShapeDtypeStruct((4, RP), f32),
                   jax.ShapeDtypeStruct((4, RP), f32)],
    )(rel_pad, k1s, k2s)


_BLK = 512
_DN = (((1,), (1,)), ((), ()))


def _prep_layer_kernel(use_deg, slot,
                       acc_ref, den_ref, rn_ref, k1_ref, b_ref, c_ref,
                       f_ref, e_ref, q_ref):
    if use_deg:
        a = acc_ref[slot]
        invd = 1.0 / jnp.maximum(den_ref[0], 1.0)
    else:
        a = acc_ref[0] + acc_ref[1]
        dn = den_ref[0] + den_ref[1]
        invd = jnp.where(dn > 0.0, 1.0 / dn, 0.0)
    fmat = jnp.tanh(a * invd[:, None])
    f_ref[...] = fmat
    g = lax.dot_general(fmat, rn_ref[...], _DN, preferred_element_type=f32)
    an = lax.dot_general(fmat, k1_ref[...], _DN, preferred_element_type=f32)
    e = jnp.exp(an + c_ref[...] - 2.0 * g * b_ref[...])
    e_ref[...] = e
    q_ref[...] = 2.0 * e * g


def _prep_layer(acc, den, rn, k1, b, c, use_deg, slot):
    grid = NP // _BLK
    outs = [jax.ShapeDtypeStruct((NP, D), f32),
            jax.ShapeDtypeStruct((NP, RP), f32),
            jax.ShapeDtypeStruct((NP, RP), f32)]
    out_specs = [pl.BlockSpec((_BLK, D), lambda i: (i, 0)),
                 pl.BlockSpec((_BLK, RP), lambda i: (i, 0)),
                 pl.BlockSpec((_BLK, RP), lambda i: (i, 0))]
    return pl.pallas_call(
        functools.partial(_prep_layer_kernel, use_deg, slot),
        grid=(grid,),
        in_specs=[pl.BlockSpec((NC, _BLK, D), lambda i: (0, i, 0)),
                  pl.BlockSpec((NC, _BLK), lambda i: (0, i)),
                  pl.BlockSpec((RP, D), lambda i: (0, 0)),
                  pl.BlockSpec((1, D), lambda i: (0, 0)),
                  pl.BlockSpec((1, RP), lambda i: (0, 0)),
                  pl.BlockSpec((1, RP), lambda i: (0, 0))],
        out_specs=out_specs,
        out_shape=outs,
    )(acc, den, rn, k1, b, c)


def _final_kernel(acc_ref, den_ref, f_ref):
    a = acc_ref[0] + acc_ref[1]
    dn = den_ref[0] + den_ref[1]
    invd = jnp.where(dn > 0.0, 1.0 / dn, 0.0)
    f_ref[...] = jnp.tanh(a * invd[:, None])


def _finalize(acc, den):
    return pl.pallas_call(
        _final_kernel,
        grid=(NP // _BLK,),
        in_specs=[pl.BlockSpec((NC, _BLK, D), lambda i: (0, i, 0)),
                  pl.BlockSpec((NC, _BLK), lambda i: (0, i))],
        out_specs=pl.BlockSpec((_BLK, D), lambda i: (i, 0)),
        out_shape=jax.ShapeDtypeStruct((NP, D), f32),
    )(acc, den)


def kernel(edge_index, edge_rel, ent_emb, rel_emb, attn_e, attn_r):
    src = edge_index[0]
    dst = edge_index[1]
    pad = EP - E
    srcp = jnp.concatenate([src, jnp.zeros((pad,), i32)])
    dstp = jnp.concatenate([dst, jnp.full((pad,), DUMMY, i32)])
    relp = jnp.concatenate([edge_rel, jnp.zeros((pad,), i32)])
    gidxp = srcp * RP + relp
    stacked = jnp.stack([srcp, dstp, relp, gidxp])
    epack0 = stacked.reshape(4, EP // CH0, CH0).transpose(1, 0, 2)
    epackl = stacked.reshape(4, EP // CHL, CHL).transpose(1, 0, 2)
    rel_pad = jnp.pad(rel_emb, ((0, RP - NREL), (0, 0)))
    ctab0 = jnp.concatenate([ent_emb, rel_emb])

    attn = [attn_e, attn_r]
    k1s = jnp.stack([attn[enc][l, 128:256, 0]
                     for enc in range(2) for l in range(2)])
    k2s = jnp.stack([attn[enc][l, 256:384, 0]
                     for enc in range(2) for l in range(2)])
    rn, btab, ctab = _prep_rel(rel_pad, k1s, k2s)

    zrows = jnp.zeros((NP, D), f32)
    zvec = jnp.zeros((NP,), f32)

    acc0, den0 = _pass0(ctab0, epack0, zrows, zvec)

    outs = []
    for enc in range(2):
        acc, den = acc0, den0
        use_deg = True
        for l in range(2):
            ki = enc * 2 + l
            fmat, emat, qmat = _prep_layer(
                acc, den, rn, k1s[ki][None], btab[ki][None], ctab[ki][None],
                use_deg, enc)
            outs.append(fmat)
            acc, den = _layer_pass(
                epackl, emat.reshape(NP * RP), qmat.reshape(NP * RP),
                fmat, rn, zrows, zvec)
            use_deg = False
        outs.append(_finalize(acc, den))

    return jnp.concatenate(outs, axis=1)[:N]
